# baseline (device time: 82477 ns/iter reference)
import jax
import jax.numpy as jnp
from jax import lax
from jax.experimental import pallas as pl
from jax.experimental.pallas import tpu as pltpu

N_DEV = 16


def kernel(x, w_mat, scale_x, scale_w):
    m_per, k_dim = x.shape
    _, n_dim = w_mat.shape
    n_per = n_dim // N_DEV
    m_glob = N_DEV * m_per

    def body(x_ref, w_ref, sx_ref, sw_ref, out_ref,
             w_buf, xb, wb, sbuf, rbuf, dma_sems, send_sems, recv_sems):
        my = lax.axis_index("i")

        barrier = pltpu.get_barrier_semaphore()
        for k in range(1, N_DEV):
            pl.semaphore_signal(
                barrier, inc=1,
                device_id=((my + k) % N_DEV,),
                device_id_type=pl.DeviceIdType.MESH,
            )
        pl.semaphore_wait(barrier, N_DEV - 1)

        xb[...] = x_ref[...].astype(jnp.bfloat16)
        s_val = sx_ref[0] * sw_ref[0]

        def w_dma(j, slot):
            col = ((my + j) % N_DEV) * n_per
            return pltpu.make_async_copy(
                w_ref.at[:, pl.ds(col, n_per)],
                w_buf.at[slot],
                dma_sems.at[slot],
            )

        def recv_chunk(k):
            src_dev = (my - k) % N_DEV
            recv = pltpu.make_async_remote_copy(
                src_ref=sbuf.at[k],
                dst_ref=rbuf.at[k],
                send_sem=send_sems.at[k],
                recv_sem=recv_sems.at[k],
                device_id=(src_dev,),
                device_id_type=pl.DeviceIdType.MESH,
            )
            recv.wait_recv()
            out_ref[pl.ds(src_dev * m_per, m_per), :] = (
                rbuf[k].astype(jnp.float32)
            )

        j_order = list(range(1, N_DEV)) + [0]
        n_slots = 3
        for t in range(min(2, N_DEV)):
            w_dma(j_order[t], t % n_slots).start()
        sends = []
        for t, j in enumerate(j_order):
            if t + 2 < N_DEV:
                w_dma(j_order[t + 2], (t + 2) % n_slots).start()
            w_dma(j, t % n_slots).wait()
            wb[...] = w_buf[t % n_slots].astype(jnp.bfloat16)
            chunk = jnp.dot(
                xb[...], wb[...], preferred_element_type=jnp.float32
            ) * s_val
            if j == 0:
                out_ref[pl.ds(my * m_per, m_per), :] = chunk
            else:
                sbuf[j, :, :] = chunk.astype(jnp.bfloat16)
                rdma = pltpu.make_async_remote_copy(
                    src_ref=sbuf.at[j],
                    dst_ref=rbuf.at[j],
                    send_sem=send_sems.at[j],
                    recv_sem=recv_sems.at[j],
                    device_id=((my + j) % N_DEV,),
                    device_id_type=pl.DeviceIdType.MESH,
                )
                rdma.start()
                sends.append(rdma)
            if t >= 2:
                recv_chunk(t - 1)
        recv_chunk(N_DEV - 1)
        for rdma in sends:
            rdma.wait_send()

    return pl.pallas_call(
        body,
        out_shape=jax.ShapeDtypeStruct((m_glob, n_per), jnp.float32),
        in_specs=[
            pl.BlockSpec(memory_space=pltpu.VMEM),
            pl.BlockSpec(memory_space=pl.ANY),
            pl.BlockSpec(memory_space=pltpu.SMEM),
            pl.BlockSpec(memory_space=pltpu.SMEM),
        ],
        out_specs=pl.BlockSpec(memory_space=pltpu.VMEM),
        scratch_shapes=[
            pltpu.VMEM((3, k_dim, n_per), jnp.float32),
            pltpu.VMEM((m_per, k_dim), jnp.bfloat16),
            pltpu.VMEM((k_dim, n_per), jnp.bfloat16),
            pltpu.VMEM((N_DEV, m_per, n_per), jnp.bfloat16),
            pltpu.VMEM((N_DEV, m_per, n_per), jnp.bfloat16),
            pltpu.SemaphoreType.DMA((3,)),
            pltpu.SemaphoreType.DMA((N_DEV,)),
            pltpu.SemaphoreType.DMA((N_DEV,)),
        ],
        compiler_params=pltpu.CompilerParams(
            collective_id=0, vmem_limit_bytes=100 * 1024 * 1024,
        ),
    )(x, w_mat, scale_x, scale_w)


# device time: 73279 ns/iter; 1.1255x vs baseline; 1.1255x over previous
import jax
import jax.numpy as jnp
from jax import lax
from jax.experimental import pallas as pl
from jax.experimental.pallas import tpu as pltpu

N_DEV = 16
N_DST = 8


def kernel(x, w_mat, scale_x, scale_w):
    m_per, k_dim = x.shape
    _, n_dim = w_mat.shape
    n_per = n_dim // N_DEV
    m_glob = N_DEV * m_per
    f8 = jnp.float8_e4m3fn

    def body(x_ref, w_ref, sx_ref, sw_ref, out_ref,
             w_buf, xbm, xbp, wb, xf8, xpeer, sbuf, rbuf,
             dma_sems, xs_sem, xr_sem, send_sems, recv_sems):
        my = lax.axis_index("i")
        r = my % 2
        partner = my + 1 - 2 * r

        barrier = pltpu.get_barrier_semaphore()
        pl.semaphore_signal(barrier, inc=1, device_id=(partner,),
                            device_id_type=pl.DeviceIdType.MESH)
        for t in range(1, N_DST):
            pl.semaphore_signal(
                barrier, inc=1,
                device_id=((my + 2 * t) % N_DEV,),
                device_id_type=pl.DeviceIdType.MESH,
            )
        pl.semaphore_wait(barrier, N_DST)

        xf8[...] = x_ref[...].astype(f8)
        x_rdma = pltpu.make_async_remote_copy(
            src_ref=xf8,
            dst_ref=xpeer,
            send_sem=xs_sem.at[0],
            recv_sem=xr_sem.at[0],
            device_id=(partner,),
            device_id_type=pl.DeviceIdType.MESH,
        )
        x_rdma.start()

        xbm[...] = x_ref[...].astype(jnp.bfloat16)
        s_val = sx_ref[0] * sw_ref[0]
        off_m = r * m_per
        off_p = (1 - r) * m_per

        def w_dma(t, slot):
            col = ((my + 2 * t) % N_DEV) * n_per
            return pltpu.make_async_copy(
                w_ref.at[:, pl.ds(col, n_per)],
                w_buf.at[slot],
                dma_sems.at[slot],
            )

        cur = w_dma(0, 0)
        cur.start()
        sends = []
        for t in range(N_DST):
            if t + 1 < N_DST:
                nxt = w_dma(t + 1, (t + 1) % 2)
                nxt.start()
            cur.wait()
            wb[...] = w_buf[t % 2].astype(jnp.bfloat16)
            mine = jnp.dot(
                xbm[...], wb[...], preferred_element_type=jnp.float32
            ) * s_val
            if t == 0:
                x_rdma.wait_recv()
                xbp[...] = xpeer[...].astype(jnp.bfloat16)
            theirs = jnp.dot(
                xbp[...], wb[...], preferred_element_type=jnp.float32
            ) * s_val
            if t == 0:
                out_ref[pl.ds(my * m_per, m_per), :] = mine
                out_ref[pl.ds(partner * m_per, m_per), :] = theirs
            else:
                sbuf[t, pl.ds(off_m, m_per), :] = mine.astype(jnp.bfloat16)
                sbuf[t, pl.ds(off_p, m_per), :] = theirs.astype(jnp.bfloat16)
                rdma = pltpu.make_async_remote_copy(
                    src_ref=sbuf.at[t],
                    dst_ref=rbuf.at[t],
                    send_sem=send_sems.at[t],
                    recv_sem=recv_sems.at[t],
                    device_id=((my + 2 * t) % N_DEV,),
                    device_id_type=pl.DeviceIdType.MESH,
                )
                rdma.start()
                sends.append(rdma)
            if t + 1 < N_DST:
                cur = nxt

        for t in range(1, N_DST):
            s_dev = (my + N_DEV - 2 * t) % N_DEV
            recv = pltpu.make_async_remote_copy(
                src_ref=sbuf.at[t],
                dst_ref=rbuf.at[t],
                send_sem=send_sems.at[t],
                recv_sem=recv_sems.at[t],
                device_id=(s_dev,),
                device_id_type=pl.DeviceIdType.MESH,
            )
            recv.wait_recv()
            out_ref[pl.ds((s_dev // 2) * 2 * m_per, 2 * m_per), :] = (
                rbuf[t].astype(jnp.float32)
            )
        for rdma in sends:
            rdma.wait_send()
        x_rdma.wait_send()

    return pl.pallas_call(
        body,
        out_shape=jax.ShapeDtypeStruct((m_glob, n_per), jnp.float32),
        in_specs=[
            pl.BlockSpec(memory_space=pltpu.VMEM),
            pl.BlockSpec(memory_space=pl.ANY),
            pl.BlockSpec(memory_space=pltpu.SMEM),
            pl.BlockSpec(memory_space=pltpu.SMEM),
        ],
        out_specs=pl.BlockSpec(memory_space=pltpu.VMEM),
        scratch_shapes=[
            pltpu.VMEM((2, k_dim, n_per), jnp.float32),
            pltpu.VMEM((m_per, k_dim), jnp.bfloat16),
            pltpu.VMEM((m_per, k_dim), jnp.bfloat16),
            pltpu.VMEM((k_dim, n_per), jnp.bfloat16),
            pltpu.VMEM((m_per, k_dim), f8),
            pltpu.VMEM((m_per, k_dim), f8),
            pltpu.VMEM((N_DST, 2 * m_per, n_per), jnp.bfloat16),
            pltpu.VMEM((N_DST, 2 * m_per, n_per), jnp.bfloat16),
            pltpu.SemaphoreType.DMA((2,)),
            pltpu.SemaphoreType.DMA((1,)),
            pltpu.SemaphoreType.DMA((1,)),
            pltpu.SemaphoreType.DMA((N_DST,)),
            pltpu.SemaphoreType.DMA((N_DST,)),
        ],
        compiler_params=pltpu.CompilerParams(
            collective_id=0, vmem_limit_bytes=100 * 1024 * 1024,
        ),
    )(x, w_mat, scale_x, scale_w)


# device time: 71408 ns/iter; 1.1550x vs baseline; 1.0262x over previous
import jax
import jax.numpy as jnp
from jax import lax
from jax.experimental import pallas as pl
from jax.experimental.pallas import tpu as pltpu

N_DEV = 16


def kernel(x, w_mat, scale_x, scale_w):
    m_per, k_dim = x.shape
    _, n_dim = w_mat.shape
    n_per = n_dim // N_DEV
    m_glob = N_DEV * m_per

    def body(x_ref, w_ref, sx_ref, sw_ref, out_ref,
             w_buf, xb, wb, sbuf, rbuf, dma_sems, send_sems, recv_sems):
        my = lax.axis_index("i")

        barrier = pltpu.get_barrier_semaphore()
        for k in range(1, N_DEV):
            pl.semaphore_signal(
                barrier, inc=1,
                device_id=((my + k) % N_DEV,),
                device_id_type=pl.DeviceIdType.MESH,
            )
        pl.semaphore_wait(barrier, N_DEV - 1)

        xb[...] = x_ref[...].astype(jnp.bfloat16)
        s_val = sx_ref[0] * sw_ref[0]

        def w_dma(j, slot):
            col = ((my + j) % N_DEV) * n_per
            return pltpu.make_async_copy(
                w_ref.at[:, pl.ds(col, n_per)],
                w_buf.at[slot],
                dma_sems.at[slot],
            )

        j_order = list(range(1, N_DEV)) + [0]
        cur = w_dma(j_order[0], 0)
        cur.start()
        sends = []
        for t, j in enumerate(j_order):
            if t + 1 < N_DEV:
                nxt = w_dma(j_order[t + 1], (t + 1) % 2)
                nxt.start()
            cur.wait()
            wb[...] = w_buf[t % 2].astype(jnp.bfloat16)
            chunk = jnp.dot(
                xb[...], wb[...], preferred_element_type=jnp.float32
            ) * s_val
            if j == 0:
                out_ref[pl.ds(my * m_per, m_per), :] = chunk
            else:
                sbuf[j, :, :] = chunk.astype(jnp.bfloat16)
                rdma = pltpu.make_async_remote_copy(
                    src_ref=sbuf.at[j],
                    dst_ref=rbuf.at[j],
                    send_sem=send_sems.at[j],
                    recv_sem=recv_sems.at[j],
                    device_id=((my + j) % N_DEV,),
                    device_id_type=pl.DeviceIdType.MESH,
                )
                rdma.start()
                sends.append(rdma)
            if t + 1 < N_DEV:
                cur = nxt

        for k in range(1, N_DEV):
            src_dev = (my - k) % N_DEV
            recv = pltpu.make_async_remote_copy(
                src_ref=sbuf.at[k],
                dst_ref=rbuf.at[k],
                send_sem=send_sems.at[k],
                recv_sem=recv_sems.at[k],
                device_id=(src_dev,),
                device_id_type=pl.DeviceIdType.MESH,
            )
            recv.wait_recv()
            out_ref[pl.ds(src_dev * m_per, m_per), :] = (
                rbuf[k].astype(jnp.float32)
            )
        for rdma in sends:
            rdma.wait_send()

    return pl.pallas_call(
        body,
        out_shape=jax.ShapeDtypeStruct((m_glob, n_per), jnp.float32),
        in_specs=[
            pl.BlockSpec(memory_space=pltpu.VMEM),
            pl.BlockSpec(memory_space=pl.ANY),
            pl.BlockSpec(memory_space=pltpu.SMEM),
            pl.BlockSpec(memory_space=pltpu.SMEM),
        ],
        out_specs=pl.BlockSpec(memory_space=pltpu.VMEM),
        scratch_shapes=[
            pltpu.VMEM((2, k_dim, n_per), jnp.float32),
            pltpu.VMEM((m_per, k_dim), jnp.bfloat16),
            pltpu.VMEM((k_dim, n_per), jnp.bfloat16),
            pltpu.VMEM((N_DEV, m_per, n_per), jnp.bfloat16),
            pltpu.VMEM((N_DEV, m_per, n_per), jnp.bfloat16),
            pltpu.SemaphoreType.DMA((2,)),
            pltpu.SemaphoreType.DMA((N_DEV,)),
            pltpu.SemaphoreType.DMA((N_DEV,)),
        ],
        compiler_params=pltpu.CompilerParams(
            collective_id=0, vmem_limit_bytes=100 * 1024 * 1024,
        ),
    )(x, w_mat, scale_x, scale_w)
